# baseline (speedup 1.0000x reference)
import functools

import jax
import jax.numpy as jnp
from jax import lax
from jax.experimental import pallas as pl
from jax.experimental.pallas import tpu as pltpu, tpu_sc as plsc

N_NODES = 10000
D = 128

NC = 2
NS = 16
NW = NC * NS

CHUNK = 128
ROWS_PER_TILE = 640
NPAD = NS * ROWS_PER_TILE
E_PER_W = 10240
K_CHUNKS = E_PER_W // CHUNK
EPAD = NW * E_PER_W
K_CORE0 = 40
K_CORE1 = 2 * K_CHUNKS - K_CORE0
PH = 8
DEG_W = 128

_MESH = plsc.VectorSubcoreMesh(core_axis_name="c", subcore_axis_name="s")


def _deg_body(col2d, ones_hbm, zeros_hbm, degp, col_v, ones_v, acc):
    c = lax.axis_index("c")
    s = lax.axis_index("s")
    rbase = s * ROWS_PER_TILE
    pltpu.sync_copy(zeros_hbm.at[pl.ds(rbase, ROWS_PER_TILE)],
                    acc.at[pl.ds(rbase, ROWS_PER_TILE)])
    pltpu.sync_copy(ones_hbm, ones_v)
    cb = (c * NS + s) * K_CHUNKS
    pltpu.sync_copy(col2d.at[pl.ds(cb, K_CHUNKS)], col_v)
    plsc.subcore_barrier()

    def body(k, _):
        pltpu.sync_copy(ones_v, acc.at[col_v.at[k]], add=True)
        return 0

    lax.fori_loop(0, K_CHUNKS, body, 0)
    plsc.subcore_barrier()
    pltpu.sync_copy(acc.at[pl.ds(rbase, ROWS_PER_TILE)],
                    degp.at[c, pl.ds(rbase, ROWS_PER_TILE)])


def _agg_body(hsrc, row2d, col2d, zeros_hbm, aggp, row_v, col_v, rows0,
              rows1, acc, sem0, sem1):
    c = lax.axis_index("c")
    s = lax.axis_index("s")
    rbase = s * ROWS_PER_TILE
    pltpu.sync_copy(zeros_hbm.at[pl.ds(rbase, ROWS_PER_TILE)],
                    acc.at[pl.ds(rbase, ROWS_PER_TILE)])
    nk = jnp.where(c == 0, K_CORE0, K_CORE1)
    cb = jnp.where(c == 0, s * K_CORE0, NS * K_CORE0 + s * K_CORE1)
    plsc.subcore_barrier()

    def phase_body(p, _):
        pltpu.sync_copy(row2d.at[pl.ds(cb + p * PH, PH)], row_v)
        pltpu.sync_copy(col2d.at[pl.ds(cb + p * PH, PH)], col_v)
        pltpu.async_copy(hsrc.at[row_v.at[0]], rows0, sem0)
        for j in range(PH // 2):
            k0 = 2 * j
            pltpu.async_copy(hsrc.at[row_v.at[k0 + 1]], rows1, sem1)
            pltpu.make_async_copy(hsrc.at[row_v.at[k0]], rows0, sem0).wait()
            pltpu.sync_copy(rows0, acc.at[col_v.at[k0]], add=True)
            knext = min(k0 + 2, PH - 1)
            pltpu.async_copy(hsrc.at[row_v.at[knext]], rows0, sem0)
            pltpu.make_async_copy(
                hsrc.at[row_v.at[k0 + 1]], rows1, sem1).wait()
            pltpu.sync_copy(rows1, acc.at[col_v.at[k0 + 1]], add=True)
        pltpu.make_async_copy(
            hsrc.at[row_v.at[PH - 1]], rows0, sem0).wait()
        return 0

    lax.fori_loop(0, nk // PH, phase_body, 0)
    plsc.subcore_barrier()
    pltpu.sync_copy(acc.at[pl.ds(rbase, ROWS_PER_TILE)],
                    aggp.at[c, pl.ds(rbase, ROWS_PER_TILE)])


def _make_deg_kernel(interpret=False):
    return pl.kernel(
        _deg_body,
        out_type=jax.ShapeDtypeStruct((NC, NPAD, DEG_W), jnp.float32),
        mesh=_MESH,
        scratch_types=[
            pltpu.VMEM((K_CHUNKS, CHUNK), jnp.int32),
            pltpu.VMEM((CHUNK, DEG_W), jnp.float32),
            pltpu.VMEM_SHARED((NPAD, DEG_W), jnp.float32),
        ],
        interpret=interpret,
    )


def _make_agg_kernel(interpret=False):
    return pl.kernel(
        _agg_body,
        out_type=jax.ShapeDtypeStruct((NC, NPAD, D), jnp.float32),
        mesh=_MESH,
        scratch_types=[
            pltpu.VMEM((PH, CHUNK), jnp.int32),
            pltpu.VMEM((PH, CHUNK), jnp.int32),
            pltpu.VMEM((CHUNK, D), jnp.float32),
            pltpu.VMEM((CHUNK, D), jnp.float32),
            pltpu.VMEM_SHARED((NPAD, D), jnp.float32),
            pltpu.SemaphoreType.DMA,
            pltpu.SemaphoreType.DMA,
        ],
        interpret=interpret,
    )


_deg_kernel = _make_deg_kernel()
_agg_kernel = _make_agg_kernel()


BLK = 1024
GRID = NPAD // BLK


def _dis_from(degp_ref):
    deg = degp_ref[0, :, 0] + degp_ref[1, :, 0] + 1.0
    return (1.0 / jnp.sqrt(deg))[:, None]


def _t1_body(x_ref, w_ref, degp_ref, o_ref):
    o_ref[...] = _dis_from(degp_ref) * jnp.dot(
        x_ref[...], w_ref[...], preferred_element_type=jnp.float32)


def _t2_body(aggp_ref, hp_ref, degp_ref, b_ref, w_ref, o_ref):
    dis = _dis_from(degp_ref)
    z = jnp.maximum(
        dis * (aggp_ref[0] + aggp_ref[1] + hp_ref[...]) + b_ref[...], 0.0)
    o_ref[...] = dis * jnp.dot(z, w_ref[...],
                               preferred_element_type=jnp.float32)


def _t3_body(aggp_ref, hp_ref, degp_ref, b_ref, o_ref):
    o_ref[...] = (_dis_from(degp_ref)
                  * (aggp_ref[0] + aggp_ref[1] + hp_ref[...]) + b_ref[...])


_ROWBLK = pl.BlockSpec((BLK, D), lambda i: (i, 0))
_WSPEC = pl.BlockSpec((D, D), lambda i: (0, 0))
_DEGSPEC = pl.BlockSpec((NC, BLK, DEG_W), lambda i: (0, i, 0))
_AGGSPEC = pl.BlockSpec((NC, BLK, D), lambda i: (0, i, 0))
_BSPEC = pl.BlockSpec((1, D), lambda i: (0, 0))
_OUT = jax.ShapeDtypeStruct((NPAD, D), jnp.float32)

_t1 = pl.pallas_call(
    _t1_body, grid=(GRID,),
    in_specs=[_ROWBLK, _WSPEC, _DEGSPEC],
    out_specs=_ROWBLK, out_shape=_OUT)

_t2 = pl.pallas_call(
    _t2_body, grid=(GRID,),
    in_specs=[_AGGSPEC, _ROWBLK, _DEGSPEC, _BSPEC, _WSPEC],
    out_specs=_ROWBLK, out_shape=_OUT)

_t3 = pl.pallas_call(
    _t3_body, grid=(GRID,),
    in_specs=[_AGGSPEC, _ROWBLK, _DEGSPEC, _BSPEC],
    out_specs=_ROWBLK, out_shape=_OUT)


@jax.jit
def kernel(x, edge_index, W1, b1, W2, b2):
    row = edge_index[0].astype(jnp.int32)
    col = edge_index[1].astype(jnp.int32)
    n_edges = row.shape[0]
    row2d = jnp.pad(row, (0, EPAD - n_edges)).reshape(EPAD // CHUNK, CHUNK)
    col2d = jnp.pad(col, (0, EPAD - n_edges),
                    constant_values=N_NODES).reshape(EPAD // CHUNK, CHUNK)
    xp = jnp.pad(x, ((0, NPAD - x.shape[0]), (0, 0)))

    ones_rows = jnp.ones((CHUNK, DEG_W), jnp.float32)
    zeros_agg = jnp.zeros((NPAD, D), jnp.float32)

    degp = _deg_kernel(col2d, ones_rows, zeros_agg)
    h1p = _t1(xp, W1, degp)
    agg1 = _agg_kernel(h1p, row2d, col2d, zeros_agg)
    h2p = _t2(agg1, h1p, degp, b1.reshape(1, D), W2)
    agg2 = _agg_kernel(h2p, row2d, col2d, zeros_agg)
    out = _t3(agg2, h2p, degp, b2.reshape(1, D))
    return out[:N_NODES]


# speedup vs baseline: 1.1567x; 1.1567x over previous
import functools

import jax
import jax.numpy as jnp
from jax import lax
from jax.experimental import pallas as pl
from jax.experimental.pallas import tpu as pltpu, tpu_sc as plsc

N_NODES = 10000
D = 128

NC = 2
NS = 16
NW = NC * NS

CHUNK = 128
ROWS_PER_TILE = 640
NPAD = NS * ROWS_PER_TILE
E_PER_W = 10240
K_CHUNKS = E_PER_W // CHUNK
H_CHUNKS = K_CHUNKS // 2
EPAD = NW * E_PER_W
DEG_W = 128

_MESH = plsc.VectorSubcoreMesh(core_axis_name="c", subcore_axis_name="s")


def _deg_body(col2d, ones_hbm, zeros_hbm, degp, col_v, ones_v, acc,
              sem0, sem1):
    c = lax.axis_index("c")
    s = lax.axis_index("s")
    rbase = s * ROWS_PER_TILE
    pltpu.sync_copy(zeros_hbm.at[pl.ds(rbase, ROWS_PER_TILE)],
                    acc.at[pl.ds(rbase, ROWS_PER_TILE)])
    pltpu.sync_copy(ones_hbm, ones_v)
    cb = (c * NS + s) * K_CHUNKS
    pltpu.sync_copy(col2d.at[pl.ds(cb, K_CHUNKS)], col_v)
    plsc.subcore_barrier()

    def body(j, _):
        k0 = 2 * j
        d0 = pltpu.async_copy(ones_v, acc.at[col_v.at[k0]], sem0, add=True)
        d1 = pltpu.async_copy(ones_v, acc.at[col_v.at[k0 + 1]], sem1,
                              add=True)
        d0.wait()
        d1.wait()
        return 0

    lax.fori_loop(0, K_CHUNKS // 2, body, 0)
    plsc.subcore_barrier()
    pltpu.sync_copy(acc.at[pl.ds(rbase, ROWS_PER_TILE)],
                    degp.at[c, pl.ds(rbase, ROWS_PER_TILE)])


def _agg_body(hsrc, row2d, col2d, zeros_hbm, aggp, row_v, col_v, rows0,
              rows1, acc, sem0, sem1):
    c = lax.axis_index("c")
    s = lax.axis_index("s")
    rbase = s * ROWS_PER_TILE
    pltpu.sync_copy(zeros_hbm.at[pl.ds(rbase, ROWS_PER_TILE)],
                    acc.at[pl.ds(rbase, ROWS_PER_TILE)])
    cb = (c * NS + s) * K_CHUNKS
    plsc.subcore_barrier()

    for phase in range(K_CHUNKS // H_CHUNKS):
        pltpu.sync_copy(row2d.at[pl.ds(cb + phase * H_CHUNKS, H_CHUNKS)],
                        row_v)
        pltpu.sync_copy(col2d.at[pl.ds(cb + phase * H_CHUNKS, H_CHUNKS)],
                        col_v)
        pltpu.async_copy(hsrc.at[row_v.at[0]], rows0, sem0)

        def body(j, _):
            k0 = 2 * j
            pltpu.async_copy(hsrc.at[row_v.at[k0 + 1]], rows1, sem1)
            pltpu.make_async_copy(hsrc.at[row_v.at[k0]], rows0, sem0).wait()
            pltpu.sync_copy(rows0, acc.at[col_v.at[k0]], add=True)
            knext = jnp.minimum(k0 + 2, H_CHUNKS - 1)
            pltpu.async_copy(hsrc.at[row_v.at[knext]], rows0, sem0)
            pltpu.make_async_copy(
                hsrc.at[row_v.at[k0 + 1]], rows1, sem1).wait()
            pltpu.sync_copy(rows1, acc.at[col_v.at[k0 + 1]], add=True)
            return 0

        lax.fori_loop(0, H_CHUNKS // 2, body, 0)
        pltpu.make_async_copy(
            hsrc.at[row_v.at[H_CHUNKS - 1]], rows0, sem0).wait()
    plsc.subcore_barrier()
    pltpu.sync_copy(acc.at[pl.ds(rbase, ROWS_PER_TILE)],
                    aggp.at[c, pl.ds(rbase, ROWS_PER_TILE)])


def _make_deg_kernel(interpret=False):
    return pl.kernel(
        _deg_body,
        out_type=jax.ShapeDtypeStruct((NC, NPAD, DEG_W), jnp.float32),
        mesh=_MESH,
        scratch_types=[
            pltpu.VMEM((K_CHUNKS, CHUNK), jnp.int32),
            pltpu.VMEM((CHUNK, DEG_W), jnp.float32),
            pltpu.VMEM_SHARED((NPAD, DEG_W), jnp.float32),
            pltpu.SemaphoreType.DMA,
            pltpu.SemaphoreType.DMA,
        ],
        interpret=interpret,
    )


def _make_agg_kernel(interpret=False):
    return pl.kernel(
        _agg_body,
        out_type=jax.ShapeDtypeStruct((NC, NPAD, D), jnp.float32),
        mesh=_MESH,
        scratch_types=[
            pltpu.VMEM((H_CHUNKS, CHUNK), jnp.int32),
            pltpu.VMEM((H_CHUNKS, CHUNK), jnp.int32),
            pltpu.VMEM((CHUNK, D), jnp.float32),
            pltpu.VMEM((CHUNK, D), jnp.float32),
            pltpu.VMEM_SHARED((NPAD, D), jnp.float32),
            pltpu.SemaphoreType.DMA,
            pltpu.SemaphoreType.DMA,
        ],
        interpret=interpret,
    )


_deg_kernel = _make_deg_kernel()
_agg_kernel = _make_agg_kernel()


BLK = 1024
GRID = NPAD // BLK


def _dis_from(degp_ref):
    deg = degp_ref[0, :, 0] + degp_ref[1, :, 0] + 1.0
    return (1.0 / jnp.sqrt(deg))[:, None]


def _t1_body(x_ref, w_ref, degp_ref, o_ref):
    o_ref[...] = _dis_from(degp_ref) * jnp.dot(
        x_ref[...], w_ref[...], preferred_element_type=jnp.float32)


def _t2_body(aggp_ref, hp_ref, degp_ref, b_ref, w_ref, o_ref):
    dis = _dis_from(degp_ref)
    z = jnp.maximum(
        dis * (aggp_ref[0] + aggp_ref[1] + hp_ref[...]) + b_ref[...], 0.0)
    o_ref[...] = dis * jnp.dot(z, w_ref[...],
                               preferred_element_type=jnp.float32)


def _t3_body(aggp_ref, hp_ref, degp_ref, b_ref, o_ref):
    o_ref[...] = (_dis_from(degp_ref)
                  * (aggp_ref[0] + aggp_ref[1] + hp_ref[...]) + b_ref[...])


_ROWBLK = pl.BlockSpec((BLK, D), lambda i: (i, 0))
_WSPEC = pl.BlockSpec((D, D), lambda i: (0, 0))
_DEGSPEC = pl.BlockSpec((NC, BLK, DEG_W), lambda i: (0, i, 0))
_AGGSPEC = pl.BlockSpec((NC, BLK, D), lambda i: (0, i, 0))
_BSPEC = pl.BlockSpec((1, D), lambda i: (0, 0))
_OUT = jax.ShapeDtypeStruct((NPAD, D), jnp.float32)

_t1 = pl.pallas_call(
    _t1_body, grid=(GRID,),
    in_specs=[_ROWBLK, _WSPEC, _DEGSPEC],
    out_specs=_ROWBLK, out_shape=_OUT)

_t2 = pl.pallas_call(
    _t2_body, grid=(GRID,),
    in_specs=[_AGGSPEC, _ROWBLK, _DEGSPEC, _BSPEC, _WSPEC],
    out_specs=_ROWBLK, out_shape=_OUT)

_t3 = pl.pallas_call(
    _t3_body, grid=(GRID,),
    in_specs=[_AGGSPEC, _ROWBLK, _DEGSPEC, _BSPEC],
    out_specs=_ROWBLK, out_shape=_OUT)


@jax.jit
def kernel(x, edge_index, W1, b1, W2, b2):
    row = edge_index[0].astype(jnp.int32)
    col = edge_index[1].astype(jnp.int32)
    n_edges = row.shape[0]
    row2d = jnp.pad(row, (0, EPAD - n_edges)).reshape(EPAD // CHUNK, CHUNK)
    col2d = jnp.pad(col, (0, EPAD - n_edges),
                    constant_values=N_NODES).reshape(EPAD // CHUNK, CHUNK)
    xp = jnp.pad(x, ((0, NPAD - x.shape[0]), (0, 0)))

    ones_rows = jnp.ones((CHUNK, DEG_W), jnp.float32)
    zeros_agg = jnp.zeros((NPAD, D), jnp.float32)

    degp = _deg_kernel(col2d, ones_rows, zeros_agg)
    h1p = _t1(xp, W1, degp)
    agg1 = _agg_kernel(h1p, row2d, col2d, zeros_agg)
    h2p = _t2(agg1, h1p, degp, b1.reshape(1, D), W2)
    agg2 = _agg_kernel(h2p, row2d, col2d, zeros_agg)
    out = _t3(agg2, h2p, degp, b2.reshape(1, D))
    return out[:N_NODES]
